# Initial kernel scaffold; baseline (speedup 1.0000x reference)
import jax
import jax.numpy as jnp
from jax.experimental import pallas as pl


def kernel(x, edge_index, W1, b1, W2, b2, W3, b3):
    raise NotImplementedError("write your pallas kernel here")


# speedup vs baseline: 17.4262x; 17.4262x over previous
import functools

import jax
import jax.numpy as jnp
from jax import lax
from jax.experimental import pallas as pl
from jax.experimental.pallas import tpu as pltpu
from jax.experimental.pallas import tpu_sc as plsc

N = 10000
E = 320000
D_IN = 128
H1 = 256
H2 = 32
C = 40

NP = 10240
CB = 128
NC = 2
NS = 16
NW = NC * NS
KPT = 80
NCH = NW * KPT
EP = NCH * CB
RPT = NP // NS
RB = 512


def _mesh():
    return plsc.VectorSubcoreMesh(core_axis_name="c", subcore_axis_name="s")


def _sc_degree(dst_r, zeros_hbm, ones_hbm):
    @functools.partial(
        pl.kernel,
        out_type=jax.ShapeDtypeStruct((NC, NP, 16), jnp.float32),
        mesh=_mesh(),
        scratch_types=[
            pltpu.VMEM((CB,), jnp.int32),
            pltpu.VMEM((CB, 16), jnp.float32),
            pltpu.VMEM_SHARED((NP, 16), jnp.float32),
        ],
    )
    def deg_kernel(dst_hbm, z_hbm, ones_hbm_ref, out_hbm, idxb, ones, acc):
        cid = lax.axis_index("c")
        sid = lax.axis_index("s")

        pltpu.sync_copy(ones_hbm_ref, ones)
        pltpu.sync_copy(
            z_hbm.at[pl.ds(sid * RPT, RPT)],
            acc.at[pl.ds(sid * RPT, RPT)],
        )
        plsc.subcore_barrier()

        base = (sid * NC + cid) * KPT

        def step(j, _):
            pltpu.sync_copy(dst_hbm.at[base + j], idxb)
            pltpu.sync_copy(ones, acc.at[idxb], add=True)
            return 0

        lax.fori_loop(0, KPT, step, 0)
        plsc.subcore_barrier()
        pltpu.sync_copy(
            acc.at[pl.ds(sid * RPT, RPT)],
            out_hbm.at[cid].at[pl.ds(sid * RPT, RPT)],
        )

    return deg_kernel(dst_r, zeros_hbm, ones_hbm)


def _sc_agg(src_r, dst_r, y, d):
    stage_y = d < 128

    scratch = [
        pltpu.VMEM((CB,), jnp.int32),
        pltpu.VMEM((CB,), jnp.int32),
        pltpu.VMEM((CB, d), jnp.float32),
        pltpu.VMEM_SHARED((NP, d), jnp.float32),
        pltpu.SemaphoreType.DMA,
    ]
    if stage_y:
        scratch.append(pltpu.VMEM_SHARED((NP, d), jnp.float32))

    @functools.partial(
        pl.kernel,
        out_type=jax.ShapeDtypeStruct((NC, NP, d), jnp.float32),
        mesh=_mesh(),
        scratch_types=scratch,
    )
    def agg_kernel(src_hbm, dst_hbm, y_hbm, z_hbm, out_hbm, sidx, didx, rows,
                   acc, sem, *maybe_ys):
        cid = lax.axis_index("c")
        sid = lax.axis_index("s")

        pltpu.sync_copy(
            z_hbm.at[pl.ds(sid * RPT, RPT)],
            acc.at[pl.ds(sid * RPT, RPT)],
        )
        if stage_y:
            ysh = maybe_ys[0]
            pltpu.sync_copy(
                y_hbm.at[pl.ds(sid * RPT, RPT)],
                ysh.at[pl.ds(sid * RPT, RPT)],
            )
            y_src = ysh
        else:
            y_src = y_hbm
        plsc.subcore_barrier()

        base = (sid * NC + cid) * KPT

        def step(j, _):
            ch = base + j
            pltpu.sync_copy(src_hbm.at[ch], sidx)
            pltpu.sync_copy(dst_hbm.at[ch], didx)
            pltpu.async_copy(y_src.at[sidx], rows, sem).wait()
            plsc.subcore_barrier()
            pltpu.sync_copy(rows, acc.at[didx], add=True)
            return 0

        lax.fori_loop(0, KPT, step, 0)
        plsc.subcore_barrier()
        pltpu.sync_copy(
            acc.at[pl.ds(sid * RPT, RPT)],
            out_hbm.at[cid].at[pl.ds(sid * RPT, RPT)],
        )

    return agg_kernel(src_r, dst_r, y, jnp.zeros((NP, d), jnp.float32))


def _tc_prep(degp, xp):
    def body(degp_ref, x_ref, dinv_ref, y_ref):
        deg = degp_ref[0, :, 0:1] + degp_ref[1, :, 0:1] + 1.0
        dinv = lax.rsqrt(deg)
        dinv_ref[...] = dinv
        y_ref[...] = x_ref[...] * dinv

    return pl.pallas_call(
        body,
        grid=(NP // RB,),
        in_specs=[
            pl.BlockSpec((NC, RB, 16), lambda i: (0, i, 0)),
            pl.BlockSpec((RB, D_IN), lambda i: (i, 0)),
        ],
        out_specs=[
            pl.BlockSpec((RB, 1), lambda i: (i, 0)),
            pl.BlockSpec((RB, D_IN), lambda i: (i, 0)),
        ],
        out_shape=[
            jax.ShapeDtypeStruct((NP, 1), jnp.float32),
            jax.ShapeDtypeStruct((NP, D_IN), jnp.float32),
        ],
    )(degp, xp)


def _tc_layer1(aggp, y1, dinv, w1, b1, w2):
    def body(a_ref, y_ref, dinv_ref, w1_ref, b1_ref, w2_ref, out_ref):
        dinv = dinv_ref[...]
        z = (a_ref[0] + a_ref[1] + y_ref[...]) * dinv
        h = jnp.dot(z, w1_ref[...], preferred_element_type=jnp.float32)
        h = jnp.maximum(h + b1_ref[...], 0.0)
        t = jnp.dot(h, w2_ref[...], preferred_element_type=jnp.float32)
        out_ref[...] = t * dinv

    return pl.pallas_call(
        body,
        grid=(NP // RB,),
        in_specs=[
            pl.BlockSpec((NC, RB, D_IN), lambda i: (0, i, 0)),
            pl.BlockSpec((RB, D_IN), lambda i: (i, 0)),
            pl.BlockSpec((RB, 1), lambda i: (i, 0)),
            pl.BlockSpec((D_IN, H1), lambda i: (0, 0)),
            pl.BlockSpec((1, H1), lambda i: (0, 0)),
            pl.BlockSpec((H1, H2), lambda i: (0, 0)),
        ],
        out_specs=pl.BlockSpec((RB, H2), lambda i: (i, 0)),
        out_shape=jax.ShapeDtypeStruct((NP, H2), jnp.float32),
    )(aggp, y1, dinv, w1, b1, w2)


def _tc_layer2(aggp, y2, dinv, b2):
    def body(a_ref, y_ref, dinv_ref, b2_ref, out_ref):
        dinv = dinv_ref[...]
        z = (a_ref[0] + a_ref[1] + y_ref[...]) * dinv
        out_ref[...] = jnp.maximum(z + b2_ref[...], 0.0) * dinv

    return pl.pallas_call(
        body,
        grid=(NP // RB,),
        in_specs=[
            pl.BlockSpec((NC, RB, H2), lambda i: (0, i, 0)),
            pl.BlockSpec((RB, H2), lambda i: (i, 0)),
            pl.BlockSpec((RB, 1), lambda i: (i, 0)),
            pl.BlockSpec((1, H2), lambda i: (0, 0)),
        ],
        out_specs=pl.BlockSpec((RB, H2), lambda i: (i, 0)),
        out_shape=jax.ShapeDtypeStruct((NP, H2), jnp.float32),
    )(aggp, y2, dinv, b2)


def _tc_layer3(aggp, y3, dinv, w3, b3):
    def body(a_ref, y_ref, dinv_ref, w3_ref, b3_ref, out_ref):
        z = (a_ref[0] + a_ref[1] + y_ref[...]) * dinv_ref[...]
        o = jnp.dot(z, w3_ref[...], preferred_element_type=jnp.float32)
        out_ref[...] = o + b3_ref[...]

    return pl.pallas_call(
        body,
        grid=(NP // RB,),
        in_specs=[
            pl.BlockSpec((NC, RB, H2), lambda i: (0, i, 0)),
            pl.BlockSpec((RB, H2), lambda i: (i, 0)),
            pl.BlockSpec((RB, 1), lambda i: (i, 0)),
            pl.BlockSpec((H2, C), lambda i: (0, 0)),
            pl.BlockSpec((1, C), lambda i: (0, 0)),
        ],
        out_specs=pl.BlockSpec((RB, C), lambda i: (i, 0)),
        out_shape=jax.ShapeDtypeStruct((NP, C), jnp.float32),
    )(aggp, y3, dinv, w3, b3)


def kernel(x, edge_index, W1, b1, W2, b2, W3, b3):
    pad = N + (jnp.arange(EP - E, dtype=jnp.int32) % (NP - N))
    src_r = jnp.concatenate([edge_index[0], pad]).reshape(NCH, CB)
    dst_r = jnp.concatenate([edge_index[1], pad]).reshape(NCH, CB)
    xp = jnp.pad(x, ((0, NP - N), (0, 0)))

    degp = _sc_degree(dst_r, jnp.zeros((NP, 16), jnp.float32),
                      jnp.ones((CB, 16), jnp.float32))
    dinv, y1 = _tc_prep(degp, xp)
    agg1 = _sc_agg(src_r, dst_r, y1, D_IN)
    y2 = _tc_layer1(agg1, y1, dinv, W1, b1.reshape(1, H1), W2)
    agg2 = _sc_agg(src_r, dst_r, y2, H2)
    y3 = _tc_layer2(agg2, y2, dinv, b2.reshape(1, H2))
    agg3 = _sc_agg(src_r, dst_r, y3, H2)
    out = _tc_layer3(agg3, y3, dinv, W3, b3.reshape(1, C))
    return out[:N]
